# baseline (speedup 1.0000x reference)
import functools

import jax
import jax.numpy as jnp
from jax import lax
from jax.experimental import pallas as pl
from jax.experimental.pallas import tpu as pltpu
from jax.experimental.pallas import tpu_sc as plsc

N_NODES = 10000
D_FEAT = 128
OUT_DIM = 128
SUPPORT = 2
N_EDGES = 320000

NC = 2
NS = 16
NW = NC * NS

E_B = 128
NBLK = 80
EPT = NBLK * E_B
E_PAD = NW * EPT
N_PAD = 10240
ROWS_PT = N_PAD // NS

MM_BM = 2000
ADD_BM = 2000


def _mm_body(x_ref, w_ref, o_ref):
    o_ref[...] = jnp.dot(x_ref[...], w_ref[0], preferred_element_type=jnp.float32)


def _add_body(p_ref, o_ref):
    o_ref[...] = p_ref[0] + p_ref[1]


def _sc_body(fw_hbm, row_hbm, col_hbm, val_hbm, part_hbm,
             acc_sh, col_v, row_v, val_v, rows_v, col_v2, row_v2, val_v2,
             rows_v2, sem, csem, sem2, csem2):
    cid = lax.axis_index("c")
    sid = lax.axis_index("s")
    wid = cid * NS + sid

    zeros16 = jnp.zeros((16,), jnp.float32)

    def zrow(i, carry):
        for j in range(8):
            rows_v[i, pl.ds(j * 16, 16)] = zeros16
        return carry

    lax.fori_loop(0, E_B, zrow, 0)
    for c in range(ROWS_PT // E_B):
        pltpu.sync_copy(rows_v, acc_sh.at[pl.ds(sid * ROWS_PT + c * E_B, E_B)])
    plsc.subcore_barrier()

    base = wid * EPT

    def scale_scatter(col_v, row_v, val_v, rows_v):
        def scale(i, c2):
            vals = val_v[pl.ds(i * 16, 16)]
            for k in range(16):
                v = vals[k]
                r = i * 16 + k
                for j in range(8):
                    sl = pl.ds(j * 16, 16)
                    rows_v[r, sl] = rows_v[r, sl] * v
            return c2

        lax.fori_loop(0, E_B // 16, scale, 0)
        pltpu.sync_copy(rows_v, acc_sh.at[row_v], add=True)

    def eblock2(i, carry):
        off0 = base + (2 * i) * E_B
        off1 = off0 + E_B
        a_col0 = pltpu.async_copy(col_hbm.at[pl.ds(off0, E_B)], col_v, csem)
        a_row0 = pltpu.async_copy(row_hbm.at[pl.ds(off0, E_B)], row_v, sem)
        a_val0 = pltpu.async_copy(val_hbm.at[pl.ds(off0, E_B)], val_v, sem)
        a_col1 = pltpu.async_copy(col_hbm.at[pl.ds(off1, E_B)], col_v2, csem2)
        a_row1 = pltpu.async_copy(row_hbm.at[pl.ds(off1, E_B)], row_v2, sem2)
        a_val1 = pltpu.async_copy(val_hbm.at[pl.ds(off1, E_B)], val_v2, sem2)
        a_col0.wait()
        g0 = pltpu.async_copy(fw_hbm.at[col_v], rows_v, sem)
        a_col1.wait()
        g1 = pltpu.async_copy(fw_hbm.at[col_v2], rows_v2, sem2)
        a_row0.wait()
        a_val0.wait()
        g0.wait()
        scale_scatter(col_v, row_v, val_v, rows_v)
        a_row1.wait()
        a_val1.wait()
        g1.wait()
        scale_scatter(col_v2, row_v2, val_v2, rows_v2)
        return carry

    lax.fori_loop(0, NBLK // 2, eblock2, 0)
    plsc.subcore_barrier()

    for c in range(ROWS_PT // E_B):
        r0 = sid * ROWS_PT + c * E_B
        pltpu.sync_copy(acc_sh.at[pl.ds(r0, E_B)], part_hbm.at[cid, pl.ds(r0, E_B)])


def kernel(x, edge_row, edge_col, edge_val, W_F):
    n, d = x.shape
    s = W_F.shape[0]
    od = W_F.shape[2]
    e = edge_row.shape[0]
    assert (n, d, od, s, e) == (N_NODES, D_FEAT, OUT_DIM, SUPPORT, N_EDGES)

    fw = pl.pallas_call(
        _mm_body,
        grid=(s, n // MM_BM),
        in_specs=[
            pl.BlockSpec((MM_BM, d), lambda si, m: (m, 0)),
            pl.BlockSpec((1, d, od), lambda si, m: (si, 0, 0)),
        ],
        out_specs=pl.BlockSpec((MM_BM, od), lambda si, m: (si * (N_NODES // MM_BM) + m, 0)),
        out_shape=jax.ShapeDtypeStruct((s * n, od), jnp.float32),
    )(x, W_F)

    mesh = plsc.VectorSubcoreMesh(core_axis_name="c", subcore_axis_name="s")
    scatter = pl.kernel(
        _sc_body,
        out_type=jax.ShapeDtypeStruct((NC, N_PAD, od), jnp.float32),
        mesh=mesh,
        scratch_types=[
            pltpu.VMEM_SHARED((N_PAD, od), jnp.float32),
            pltpu.VMEM((E_B,), jnp.int32),
            pltpu.VMEM((E_B,), jnp.int32),
            pltpu.VMEM((E_B,), jnp.float32),
            pltpu.VMEM((E_B, od), jnp.float32),
            pltpu.VMEM((E_B,), jnp.int32),
            pltpu.VMEM((E_B,), jnp.int32),
            pltpu.VMEM((E_B,), jnp.float32),
            pltpu.VMEM((E_B, od), jnp.float32),
            pltpu.SemaphoreType.DMA,
            pltpu.SemaphoreType.DMA,
            pltpu.SemaphoreType.DMA,
            pltpu.SemaphoreType.DMA,
        ],
    )
    pad = E_PAD - e
    part = scatter(fw, jnp.pad(edge_row, (0, pad)), jnp.pad(edge_col, (0, pad)),
                   jnp.pad(edge_val, (0, pad)))

    out = pl.pallas_call(
        _add_body,
        grid=(n // ADD_BM,),
        in_specs=[pl.BlockSpec((NC, ADD_BM, od), lambda m: (0, m, 0))],
        out_specs=pl.BlockSpec((ADD_BM, od), lambda m: (m, 0)),
        out_shape=jax.ShapeDtypeStruct((n, od), jnp.float32),
    )(part)
    return out


# speedup vs baseline: 1.8473x; 1.8473x over previous
import functools

import jax
import jax.numpy as jnp
from jax import lax
from jax.experimental import pallas as pl
from jax.experimental.pallas import tpu as pltpu
from jax.experimental.pallas import tpu_sc as plsc

N_NODES = 10000
D_FEAT = 128
OUT_DIM = 128
SUPPORT = 2
N_EDGES = 320000

NC = 2
NS = 16
NW = NC * NS

E_B = 80
NBLK = 126
EPT = NBLK * E_B
E_PAD = NW * EPT
N_PAD = 10240
ROWS_PT = N_PAD // NS
ZR = 128

MM_BM = 2000
ADD_BM = 2000


def _mm_body(x_ref, w_ref, o_ref):
    o_ref[...] = jnp.dot(x_ref[...], w_ref[0], preferred_element_type=jnp.float32)


def _add_body(p_ref, o_ref):
    o_ref[...] = p_ref[0] + p_ref[1]


def _sc_body(fw_hbm, row_hbm, col_hbm, val_hbm, part_hbm,
             acc_sh, col_v, row_v, val_v, rows_v, col_v2, row_v2, val_v2,
             rows_v2, col_v3, row_v3, val_v3, rows_v3, zbuf,
             sem, csem, sem2, csem2, sem3, csem3):
    cid = lax.axis_index("c")
    sid = lax.axis_index("s")
    wid = cid * NS + sid

    zeros16 = jnp.zeros((16,), jnp.float32)

    def zrow(i, carry):
        for j in range(8):
            zbuf[i, pl.ds(j * 16, 16)] = zeros16
        return carry

    lax.fori_loop(0, ZR, zrow, 0)
    for c in range(ROWS_PT // ZR):
        pltpu.sync_copy(zbuf, acc_sh.at[pl.ds(sid * ROWS_PT + c * ZR, ZR)])
    plsc.subcore_barrier()

    base = wid * EPT

    def scale(val_v, rows_v):
        def body(i, c2):
            vals = val_v[pl.ds(i * 16, 16)]
            for k in range(16):
                v = vals[k]
                r = i * 16 + k
                for j in range(8):
                    sl = pl.ds(j * 16, 16)
                    rows_v[r, sl] = rows_v[r, sl] * v
            return c2

        lax.fori_loop(0, E_B // 16, body, 0)

    def eblock3(i, carry):
        off0 = base + (3 * i) * E_B
        off1 = off0 + E_B
        off2 = off1 + E_B
        a_col0 = pltpu.async_copy(col_hbm.at[pl.ds(off0, E_B)], col_v, csem)
        a_row0 = pltpu.async_copy(row_hbm.at[pl.ds(off0, E_B)], row_v, sem)
        a_val0 = pltpu.async_copy(val_hbm.at[pl.ds(off0, E_B)], val_v, sem)
        a_col1 = pltpu.async_copy(col_hbm.at[pl.ds(off1, E_B)], col_v2, csem2)
        a_row1 = pltpu.async_copy(row_hbm.at[pl.ds(off1, E_B)], row_v2, sem2)
        a_val1 = pltpu.async_copy(val_hbm.at[pl.ds(off1, E_B)], val_v2, sem2)
        a_col2 = pltpu.async_copy(col_hbm.at[pl.ds(off2, E_B)], col_v3, csem3)
        a_row2 = pltpu.async_copy(row_hbm.at[pl.ds(off2, E_B)], row_v3, sem3)
        a_val2 = pltpu.async_copy(val_hbm.at[pl.ds(off2, E_B)], val_v3, sem3)
        a_col0.wait()
        g0 = pltpu.async_copy(fw_hbm.at[col_v], rows_v, sem)
        a_col1.wait()
        g1 = pltpu.async_copy(fw_hbm.at[col_v2], rows_v2, sem2)
        a_col2.wait()
        g2 = pltpu.async_copy(fw_hbm.at[col_v3], rows_v3, sem3)
        a_row0.wait()
        a_val0.wait()
        g0.wait()
        scale(val_v, rows_v)
        s0 = pltpu.async_copy(rows_v, acc_sh.at[row_v], sem, add=True)
        a_row1.wait()
        a_val1.wait()
        g1.wait()
        scale(val_v2, rows_v2)
        s1 = pltpu.async_copy(rows_v2, acc_sh.at[row_v2], sem2, add=True)
        a_row2.wait()
        a_val2.wait()
        g2.wait()
        scale(val_v3, rows_v3)
        s2 = pltpu.async_copy(rows_v3, acc_sh.at[row_v3], sem3, add=True)
        s0.wait()
        s1.wait()
        s2.wait()
        return carry

    lax.fori_loop(0, NBLK // 3, eblock3, 0)
    plsc.subcore_barrier()

    for c in range(ROWS_PT // ZR):
        r0 = sid * ROWS_PT + c * ZR
        pltpu.sync_copy(acc_sh.at[pl.ds(r0, ZR)], part_hbm.at[cid, pl.ds(r0, ZR)])


def kernel(x, edge_row, edge_col, edge_val, W_F):
    n, d = x.shape
    s = W_F.shape[0]
    od = W_F.shape[2]
    e = edge_row.shape[0]
    assert (n, d, od, s, e) == (N_NODES, D_FEAT, OUT_DIM, SUPPORT, N_EDGES)

    fw = pl.pallas_call(
        _mm_body,
        grid=(s, n // MM_BM),
        in_specs=[
            pl.BlockSpec((MM_BM, d), lambda si, m: (m, 0)),
            pl.BlockSpec((1, d, od), lambda si, m: (si, 0, 0)),
        ],
        out_specs=pl.BlockSpec((MM_BM, od), lambda si, m: (si * (N_NODES // MM_BM) + m, 0)),
        out_shape=jax.ShapeDtypeStruct((s * n, od), jnp.float32),
    )(x, W_F)

    mesh = plsc.VectorSubcoreMesh(core_axis_name="c", subcore_axis_name="s")
    scatter = pl.kernel(
        _sc_body,
        out_type=jax.ShapeDtypeStruct((NC, N_PAD, od), jnp.float32),
        mesh=mesh,
        scratch_types=[
            pltpu.VMEM_SHARED((N_PAD, od), jnp.float32),
            pltpu.VMEM((E_B,), jnp.int32),
            pltpu.VMEM((E_B,), jnp.int32),
            pltpu.VMEM((E_B,), jnp.float32),
            pltpu.VMEM((E_B, od), jnp.float32),
            pltpu.VMEM((E_B,), jnp.int32),
            pltpu.VMEM((E_B,), jnp.int32),
            pltpu.VMEM((E_B,), jnp.float32),
            pltpu.VMEM((E_B, od), jnp.float32),
            pltpu.VMEM((E_B,), jnp.int32),
            pltpu.VMEM((E_B,), jnp.int32),
            pltpu.VMEM((E_B,), jnp.float32),
            pltpu.VMEM((E_B, od), jnp.float32),
            pltpu.VMEM((ZR, od), jnp.float32),
            pltpu.SemaphoreType.DMA,
            pltpu.SemaphoreType.DMA,
            pltpu.SemaphoreType.DMA,
            pltpu.SemaphoreType.DMA,
            pltpu.SemaphoreType.DMA,
            pltpu.SemaphoreType.DMA,
        ],
    )
    pad = E_PAD - e
    part = scatter(fw, jnp.pad(edge_row, (0, pad)), jnp.pad(edge_col, (0, pad)),
                   jnp.pad(edge_val, (0, pad)))

    out = pl.pallas_call(
        _add_body,
        grid=(n // ADD_BM,),
        in_specs=[pl.BlockSpec((NC, ADD_BM, od), lambda m: (0, m, 0))],
        out_specs=pl.BlockSpec((ADD_BM, od), lambda m: (m, 0)),
        out_shape=jax.ShapeDtypeStruct((n, od), jnp.float32),
    )(part)
    return out


# speedup vs baseline: 2.5513x; 1.3811x over previous
import functools

import jax
import jax.numpy as jnp
from jax import lax
from jax.experimental import pallas as pl
from jax.experimental.pallas import tpu as pltpu
from jax.experimental.pallas import tpu_sc as plsc

N_NODES = 10000
D_FEAT = 128
OUT_DIM = 128
SUPPORT = 2
N_EDGES = 320000

NC = 2
NS = 16
NW = NC * NS

EPT = N_EDGES // NW
E_B = 80
NBLK = EPT // E_B
N_PAD = 10240
ROWS_PT = N_PAD // NS
ZR = 128

MM_BM = 2000
ADD_BM = 2000


def _mm_body(x_ref, w_ref, o_ref):
    o_ref[...] = jnp.dot(x_ref[...], w_ref[0], preferred_element_type=jnp.float32)


def _add_body(p_ref, o_ref):
    o_ref[...] = p_ref[0] + p_ref[1]


def _sc_body(fw_hbm, row_hbm, col_hbm, val_hbm, part_hbm,
             acc_sh, col_v, row_v, val_v, rows_v, col_v2, row_v2, val_v2,
             rows_v2, zbuf, sem, csem, sem2, csem2):
    cid = lax.axis_index("c")
    sid = lax.axis_index("s")
    wid = cid * NS + sid

    zeros16 = jnp.zeros((16,), jnp.float32)

    def zrow(i, carry):
        for j in range(8):
            zbuf[i, pl.ds(j * 16, 16)] = zeros16
        return carry

    lax.fori_loop(0, ZR, zrow, 0)
    for c in range(ROWS_PT // ZR):
        pltpu.sync_copy(zbuf, acc_sh.at[pl.ds(sid * ROWS_PT + c * ZR, ZR)])
    plsc.subcore_barrier()

    base = wid * EPT

    def scale_scatter(col_v, row_v, val_v, rows_v):
        def scale(i, c2):
            vals = val_v[pl.ds(i * 16, 16)]
            for k in range(16):
                v = vals[k]
                r = i * 16 + k
                for j in range(8):
                    sl = pl.ds(j * 16, 16)
                    rows_v[r, sl] = rows_v[r, sl] * v
            return c2

        lax.fori_loop(0, E_B // 16, scale, 0)

    def eblock2(i, carry):
        off0 = base + (2 * i) * E_B
        off1 = off0 + E_B
        a_col0 = pltpu.async_copy(col_hbm.at[pl.ds(off0, E_B)], col_v, csem)
        a_row0 = pltpu.async_copy(row_hbm.at[pl.ds(off0, E_B)], row_v, sem)
        a_val0 = pltpu.async_copy(val_hbm.at[pl.ds(off0, E_B)], val_v, sem)
        a_col1 = pltpu.async_copy(col_hbm.at[pl.ds(off1, E_B)], col_v2, csem2)
        a_row1 = pltpu.async_copy(row_hbm.at[pl.ds(off1, E_B)], row_v2, sem2)
        a_val1 = pltpu.async_copy(val_hbm.at[pl.ds(off1, E_B)], val_v2, sem2)
        a_col0.wait()
        g0 = pltpu.async_copy(fw_hbm.at[col_v], rows_v, sem)
        a_col1.wait()
        g1 = pltpu.async_copy(fw_hbm.at[col_v2], rows_v2, sem2)
        a_row0.wait()
        a_val0.wait()
        g0.wait()
        scale_scatter(col_v, row_v, val_v, rows_v)
        s0 = pltpu.async_copy(rows_v, acc_sh.at[row_v], sem, add=True)
        a_row1.wait()
        a_val1.wait()
        g1.wait()
        scale_scatter(col_v2, row_v2, val_v2, rows_v2)
        s1 = pltpu.async_copy(rows_v2, acc_sh.at[row_v2], sem2, add=True)
        s0.wait()
        s1.wait()
        return carry

    lax.fori_loop(0, NBLK // 2, eblock2, 0)

    offl = base + (NBLK - 1) * E_B
    a_col = pltpu.async_copy(col_hbm.at[pl.ds(offl, E_B)], col_v, csem)
    a_row = pltpu.async_copy(row_hbm.at[pl.ds(offl, E_B)], row_v, sem)
    a_val = pltpu.async_copy(val_hbm.at[pl.ds(offl, E_B)], val_v, sem)
    a_col.wait()
    g = pltpu.async_copy(fw_hbm.at[col_v], rows_v, sem)
    a_row.wait()
    a_val.wait()
    g.wait()
    scale_scatter(col_v, row_v, val_v, rows_v)
    pltpu.sync_copy(rows_v, acc_sh.at[row_v], add=True)
    plsc.subcore_barrier()

    for c in range(ROWS_PT // ZR):
        r0 = sid * ROWS_PT + c * ZR
        pltpu.sync_copy(acc_sh.at[pl.ds(r0, ZR)], part_hbm.at[cid, pl.ds(r0, ZR)])


def kernel(x, edge_row, edge_col, edge_val, W_F):
    n, d = x.shape
    s = W_F.shape[0]
    od = W_F.shape[2]
    e = edge_row.shape[0]
    assert (n, d, od, s, e) == (N_NODES, D_FEAT, OUT_DIM, SUPPORT, N_EDGES)

    fw = pl.pallas_call(
        _mm_body,
        grid=(s, n // MM_BM),
        in_specs=[
            pl.BlockSpec((MM_BM, d), lambda si, m: (m, 0)),
            pl.BlockSpec((1, d, od), lambda si, m: (si, 0, 0)),
        ],
        out_specs=pl.BlockSpec((MM_BM, od), lambda si, m: (si * (N_NODES // MM_BM) + m, 0)),
        out_shape=jax.ShapeDtypeStruct((s * n, od), jnp.float32),
    )(x, W_F)

    mesh = plsc.VectorSubcoreMesh(core_axis_name="c", subcore_axis_name="s")
    scatter = pl.kernel(
        _sc_body,
        out_type=jax.ShapeDtypeStruct((NC, N_PAD, od), jnp.float32),
        mesh=mesh,
        scratch_types=[
            pltpu.VMEM_SHARED((N_PAD, od), jnp.float32),
            pltpu.VMEM((E_B,), jnp.int32),
            pltpu.VMEM((E_B,), jnp.int32),
            pltpu.VMEM((E_B,), jnp.float32),
            pltpu.VMEM((E_B, od), jnp.float32),
            pltpu.VMEM((E_B,), jnp.int32),
            pltpu.VMEM((E_B,), jnp.int32),
            pltpu.VMEM((E_B,), jnp.float32),
            pltpu.VMEM((E_B, od), jnp.float32),
            pltpu.VMEM((ZR, od), jnp.float32),
            pltpu.SemaphoreType.DMA,
            pltpu.SemaphoreType.DMA,
            pltpu.SemaphoreType.DMA,
            pltpu.SemaphoreType.DMA,
        ],
    )
    part = scatter(fw, edge_row, edge_col, edge_val)

    out = pl.pallas_call(
        _add_body,
        grid=(n // ADD_BM,),
        in_specs=[pl.BlockSpec((NC, ADD_BM, od), lambda m: (0, m, 0))],
        out_specs=pl.BlockSpec((ADD_BM, od), lambda m: (m, 0)),
        out_shape=jax.ShapeDtypeStruct((n, od), jnp.float32),
    )(part)
    return out
